# Initial kernel scaffold; baseline (speedup 1.0000x reference)
import functools

import jax
import jax.numpy as jnp
from jax import lax
from jax.experimental import pallas as pl
from jax.experimental.pallas import tpu as pltpu
from jax.experimental.pallas import tpu_sc as plsc

_N = 10000
_E = 320000
_A = 100
_D = 128
_B = 64
_OUT = 10

_NC = 2
_NS = 16
_PT = _E // (_NC * _NS)
_CH = 80
_NCHUNK = _PT // _CH
_RPT = _N // _NS
_RZ = 125


def _edge_body(ei_hbm, u_hbm, vx_hbm, agg_hbm,
               srcv, dstv, ubuf, vbuf, zbuf, aggs, sem):
    c = lax.axis_index("c")
    s = lax.axis_index("s")

    def zrow(r, carry):
        for j in range(_D // 16):
            zbuf[r, pl.ds(j * 16, 16)] = jnp.zeros((16,), jnp.float32)
        return carry
    lax.fori_loop(0, _RZ, zrow, 0)
    for j in range(_RPT // _RZ):
        pltpu.sync_copy(zbuf, aggs.at[pl.ds(s * _RPT + j * _RZ, _RZ)])
    plsc.subcore_barrier()

    base0 = (c * _NS + s) * _PT

    def chunk(i, carry):
        base = base0 + i * _CH
        pltpu.sync_copy(ei_hbm.at[0, pl.ds(base, _CH)], srcv)
        pltpu.sync_copy(ei_hbm.at[1, pl.ds(base, _CH)], dstv)
        pltpu.async_copy(u_hbm.at[srcv], ubuf, sem).wait()
        pltpu.async_copy(vx_hbm.at[dstv], vbuf, sem).wait()

        def row(r, rc):
            for j in range(_D // 16):
                a = ubuf[r, pl.ds(j * 16, 16)]
                b = vbuf[r, pl.ds(j * 16, 16)]
                ubuf[r, pl.ds(j * 16, 16)] = jnp.maximum(a + b, 0.0)
            return rc
        lax.fori_loop(0, _CH, row, 0)

        pltpu.sync_copy(ubuf, aggs.at[dstv], add=True)
        return carry
    lax.fori_loop(0, _NCHUNK, chunk, 0)

    plsc.subcore_barrier()
    for j in range(_RPT // _RZ):
        rows = pl.ds(s * _RPT + j * _RZ, _RZ)
        pltpu.sync_copy(aggs.at[rows], agg_hbm.at[c, rows])


def _make_edge_kernel():
    mesh = plsc.VectorSubcoreMesh(core_axis_name="c", subcore_axis_name="s")
    return pl.kernel(
        _edge_body,
        out_type=jax.ShapeDtypeStruct((_NC, _N, _D), jnp.float32),
        mesh=mesh,
        scratch_types=[
            pltpu.VMEM((_CH,), jnp.int32),
            pltpu.VMEM((_CH,), jnp.int32),
            pltpu.VMEM((_CH, _D), jnp.float32),
            pltpu.VMEM((_CH, _D), jnp.float32),
            pltpu.VMEM((_RZ, _D), jnp.float32),
            pltpu.VMEM_SHARED((_N, _D), jnp.float32),
            pltpu.SemaphoreType.DMA,
        ],
    )


def _embed_body(xidx_ref, emb_ref, vx_ref, u_ref):
    ids = xidx_ref[...]
    io = lax.broadcasted_iota(jnp.int32, (_N, _A), 1)
    oh = jnp.where(io == ids, 1.0, 0.0).astype(jnp.float32)
    vx = jnp.dot(oh, emb_ref[...], preferred_element_type=jnp.float32)
    vx_ref[...] = vx
    u_ref[...] = vx * 2.0


def _bn_block(y, g, b):
    m = jnp.mean(y, axis=0, keepdims=True)
    yc = y - m
    v = jnp.mean(yc * yc, axis=0, keepdims=True)
    return yc * lax.rsqrt(v + 1e-5) * g + b


def _mlp_body(x_ref, agg_ref, vx_ref, w1_ref, b1_ref, g1_ref, t1_ref,
              w2_ref, b2_ref, g2_ref, t2_ref, xo_ref, uo_ref):
    h = x_ref[...] + agg_ref[0] + agg_ref[1]
    y = jnp.dot(h, w1_ref[...], preferred_element_type=jnp.float32)
    y = jnp.maximum(_bn_block(y + b1_ref[...], g1_ref[...], t1_ref[...]), 0.0)
    z = jnp.dot(y, w2_ref[...], preferred_element_type=jnp.float32)
    z = jnp.maximum(_bn_block(z + b2_ref[...], g2_ref[...], t2_ref[...]), 0.0)
    xo_ref[...] = z
    uo_ref[...] = z + vx_ref[...]


def _pool_body(x_ref, batch_ref, w1_ref, b1_ref, w2_ref, b2_ref, out_ref):
    bt = batch_ref[...]
    io = lax.broadcasted_iota(jnp.int32, (_B, _N), 0)
    oh = jnp.where(io == bt, 1.0, 0.0).astype(jnp.float32)
    pooled = jnp.dot(oh, x_ref[...], preferred_element_type=jnp.float32)
    hh = jnp.maximum(
        jnp.dot(pooled, w1_ref[...], preferred_element_type=jnp.float32)
        + b1_ref[...], 0.0)
    out_ref[...] = (
        jnp.dot(hh, w2_ref[...], preferred_element_type=jnp.float32)
        + b2_ref[...])


@jax.jit
def kernel(x_idx, edge_index, batch, emb, convs, lin1_W, lin1_b, lin2_W,
           lin2_b):
    f32 = jnp.float32
    embed = pl.pallas_call(
        _embed_body,
        out_shape=[jax.ShapeDtypeStruct((_N, _D), f32),
                   jax.ShapeDtypeStruct((_N, _D), f32)],
    )
    vx, u = embed(x_idx, emb)

    edge_fn = _make_edge_kernel()
    mlp = pl.pallas_call(
        _mlp_body,
        out_shape=[jax.ShapeDtypeStruct((_N, _D), f32),
                   jax.ShapeDtypeStruct((_N, _D), f32)],
    )

    x = vx
    for p in convs:
        aggs = edge_fn(edge_index, u, vx)
        x, u = mlp(x, aggs, vx,
                   p['W1'], p['b1'].reshape(1, -1), p['g1'].reshape(1, -1),
                   p['bt1'].reshape(1, -1),
                   p['W2'], p['b2'].reshape(1, -1), p['g2'].reshape(1, -1),
                   p['bt2'].reshape(1, -1))

    pool = pl.pallas_call(
        _pool_body,
        out_shape=jax.ShapeDtypeStruct((_B, _OUT), f32),
    )
    return pool(x, batch.reshape(1, -1), lin1_W, lin1_b.reshape(1, -1),
                lin2_W, lin2_b.reshape(1, -1))


# speedup vs baseline: 3.2864x; 3.2864x over previous
import functools

import jax
import jax.numpy as jnp
from jax import lax
from jax.experimental import pallas as pl
from jax.experimental.pallas import tpu as pltpu
from jax.experimental.pallas import tpu_sc as plsc

_N = 10000
_E = 320000
_A = 100
_D = 128
_B = 64
_OUT = 10

_NC = 2
_NS = 16
_PT = _E // (_NC * _NS)
_CH = 80
_NCHUNK = _PT // _CH
_RB = 80
_NRB = _N // _RB


def _edge_body(src_hbm, dst_hbm, u_hbm, vx_hbm, agg_hbm,
               srcv, dstv, ubuf, vbuf, zbuf, aggs, sem):
    c = lax.axis_index("c")
    s = lax.axis_index("s")

    def zrow(r, carry):
        for j in range(_D // 16):
            zbuf[r, pl.ds(j * 16, 16)] = jnp.zeros((16,), jnp.float32)
        return carry
    lax.fori_loop(0, _RB, zrow, 0)

    def zcopy(k, carry):
        blk = s + _NS * k

        @pl.when(blk < _NRB)
        def _():
            pltpu.sync_copy(zbuf, aggs.at[pl.ds(blk * _RB, _RB)])
        return carry
    lax.fori_loop(0, (_NRB + _NS - 1) // _NS, zcopy, 0)
    plsc.subcore_barrier()

    base0 = (c * _NS + s) * _PT

    def chunk(i, carry):
        base = base0 + i * _CH
        pltpu.sync_copy(src_hbm.at[pl.ds(base, _CH)], srcv)
        pltpu.sync_copy(dst_hbm.at[pl.ds(base, _CH)], dstv)
        pltpu.async_copy(u_hbm.at[srcv], ubuf, sem).wait()
        pltpu.async_copy(vx_hbm.at[dstv], vbuf, sem).wait()

        def row(r, rc):
            for j in range(_D // 16):
                a = ubuf[r, pl.ds(j * 16, 16)]
                b = vbuf[r, pl.ds(j * 16, 16)]
                ubuf[r, pl.ds(j * 16, 16)] = jnp.maximum(a + b, 0.0)
            return rc
        lax.fori_loop(0, _CH, row, 0)

        pltpu.sync_copy(ubuf, aggs.at[dstv], add=True)
        return carry
    lax.fori_loop(0, _NCHUNK, chunk, 0)

    plsc.subcore_barrier()

    def wcopy(k, carry):
        blk = s + _NS * k

        @pl.when(blk < _NRB)
        def _():
            rows = pl.ds(blk * _RB, _RB)
            pltpu.sync_copy(aggs.at[rows], agg_hbm.at[c, rows])
        return carry
    lax.fori_loop(0, (_NRB + _NS - 1) // _NS, wcopy, 0)


def _make_edge_kernel():
    mesh = plsc.VectorSubcoreMesh(core_axis_name="c", subcore_axis_name="s")
    return pl.kernel(
        _edge_body,
        out_type=jax.ShapeDtypeStruct((_NC, _N, _D), jnp.float32),
        mesh=mesh,
        scratch_types=[
            pltpu.VMEM((_CH,), jnp.int32),
            pltpu.VMEM((_CH,), jnp.int32),
            pltpu.VMEM((_CH, _D), jnp.float32),
            pltpu.VMEM((_CH, _D), jnp.float32),
            pltpu.VMEM((_RB, _D), jnp.float32),
            pltpu.VMEM_SHARED((_N, _D), jnp.float32),
            pltpu.SemaphoreType.DMA,
        ],
    )


def _embed_body(xidx_ref, emb_ref, vx_ref, u_ref):
    ids = xidx_ref[...]
    io = lax.broadcasted_iota(jnp.int32, (_N, _A), 1)
    oh = jnp.where(io == ids, 1.0, 0.0).astype(jnp.float32)
    vx = jnp.dot(oh, emb_ref[...], preferred_element_type=jnp.float32)
    vx_ref[...] = vx
    u_ref[...] = vx * 2.0


def _bn_block(y, g, b):
    m = jnp.mean(y, axis=0, keepdims=True)
    yc = y - m
    v = jnp.mean(yc * yc, axis=0, keepdims=True)
    return yc * lax.rsqrt(v + 1e-5) * g + b


def _mlp_body(x_ref, agg_ref, vx_ref, w1_ref, b1_ref, g1_ref, t1_ref,
              w2_ref, b2_ref, g2_ref, t2_ref, xo_ref, uo_ref):
    h = x_ref[...] + agg_ref[0] + agg_ref[1]
    y = jnp.dot(h, w1_ref[...], preferred_element_type=jnp.float32)
    y = jnp.maximum(_bn_block(y + b1_ref[...], g1_ref[...], t1_ref[...]), 0.0)
    z = jnp.dot(y, w2_ref[...], preferred_element_type=jnp.float32)
    z = jnp.maximum(_bn_block(z + b2_ref[...], g2_ref[...], t2_ref[...]), 0.0)
    xo_ref[...] = z
    uo_ref[...] = z + vx_ref[...]


def _pool_body(x_ref, batch_ref, w1_ref, b1_ref, w2_ref, b2_ref, out_ref):
    bt = batch_ref[...]
    io = lax.broadcasted_iota(jnp.int32, (_B, _N), 0)
    oh = jnp.where(io == bt, 1.0, 0.0).astype(jnp.float32)
    pooled = jnp.dot(oh, x_ref[...], preferred_element_type=jnp.float32)
    hh = jnp.maximum(
        jnp.dot(pooled, w1_ref[...], preferred_element_type=jnp.float32)
        + b1_ref[...], 0.0)
    out_ref[...] = (
        jnp.dot(hh, w2_ref[...], preferred_element_type=jnp.float32)
        + b2_ref[...])


@jax.jit
def kernel(x_idx, edge_index, batch, emb, convs, lin1_W, lin1_b, lin2_W,
           lin2_b):
    f32 = jnp.float32
    embed = pl.pallas_call(
        _embed_body,
        out_shape=[jax.ShapeDtypeStruct((_N, _D), f32),
                   jax.ShapeDtypeStruct((_N, _D), f32)],
    )
    vx, u = embed(x_idx, emb)

    edge_fn = _make_edge_kernel()
    mlp = pl.pallas_call(
        _mlp_body,
        out_shape=[jax.ShapeDtypeStruct((_N, _D), f32),
                   jax.ShapeDtypeStruct((_N, _D), f32)],
    )

    src = edge_index[0]
    dst = edge_index[1]
    x = vx
    for p in convs:
        aggs = edge_fn(src, dst, u, vx)
        x, u = mlp(x, aggs, vx,
                   p['W1'], p['b1'].reshape(1, -1), p['g1'].reshape(1, -1),
                   p['bt1'].reshape(1, -1),
                   p['W2'], p['b2'].reshape(1, -1), p['g2'].reshape(1, -1),
                   p['bt2'].reshape(1, -1))

    pool = pl.pallas_call(
        _pool_body,
        out_shape=jax.ShapeDtypeStruct((_B, _OUT), f32),
    )
    return pool(x, batch.reshape(1, -1), lin1_W, lin1_b.reshape(1, -1),
                lin2_W, lin2_b.reshape(1, -1))


# speedup vs baseline: 8.8714x; 2.6995x over previous
import functools

import jax
import jax.numpy as jnp
from jax import lax
from jax.experimental import pallas as pl
from jax.experimental.pallas import tpu as pltpu
from jax.experimental.pallas import tpu_sc as plsc

_N = 10000
_E = 320000
_A = 100
_D = 128
_B = 64
_OUT = 10

_NC = 2
_NS = 16
_PT = _E // (_NC * _NS)
_CH = 40
_NCHUNK = _PT // _CH
_NBUF = 3
_NOUTER = (_NCHUNK + _NBUF - 1) // _NBUF
_RB = 40
_NRB = _N // _RB


def _edge_body(src_hbm, dst_hbm, u_hbm, vx_hbm, agg_hbm,
               s0_src, s0_dst, s0_dw, s0_u, s0_v, s0_m,
               s1_src, s1_dst, s1_dw, s1_u, s1_v, s1_m,
               s2_src, s2_dst, s2_dw, s2_u, s2_v, s2_m,
               aggs,
               isem0, isem1, isem2, gsem0, gsem1, gsem2,
               ssem0, ssem1, ssem2):
    c = lax.axis_index("c")
    s = lax.axis_index("s")
    srcv = (s0_src, s1_src, s2_src)
    dstv = (s0_dst, s1_dst, s2_dst)
    dws = (s0_dw, s1_dw, s2_dw)
    ubufs = (s0_u, s1_u, s2_u)
    vbufs = (s0_v, s1_v, s2_v)
    mbufs = (s0_m, s1_m, s2_m)
    isems = (isem0, isem1, isem2)
    gsems = (gsem0, gsem1, gsem2)
    ssems = (ssem0, ssem1, ssem2)

    def zrow(r, carry):
        for j in range(_D // 16):
            s0_m[r, pl.ds(j * 16, 16)] = jnp.zeros((16,), jnp.float32)
        return carry
    lax.fori_loop(0, _RB, zrow, 0)

    def zcopy(k, carry):
        blk = s + _NS * k

        @pl.when(blk < _NRB)
        def _():
            pltpu.sync_copy(s0_m, aggs.at[pl.ds(blk * _RB, _RB)])
        return carry
    lax.fori_loop(0, (_NRB + _NS - 1) // _NS, zcopy, 0)
    plsc.subcore_barrier()

    base0 = (c * _NS + s) * _PT

    def idx_issue(j, b):
        off = base0 + j * _CH
        pltpu.async_copy(src_hbm.at[pl.ds(off, _CH)], srcv[b], isems[b])
        pltpu.async_copy(dst_hbm.at[pl.ds(off, _CH)], dstv[b], isems[b])

    def idx_wait(b):
        pltpu.make_async_copy(src_hbm.at[pl.ds(0, _CH)], srcv[b],
                              isems[b]).wait()
        pltpu.make_async_copy(dst_hbm.at[pl.ds(0, _CH)], dstv[b],
                              isems[b]).wait()

    def gathers_issue(b):
        pltpu.async_copy(u_hbm.at[srcv[b]], ubufs[b], gsems[b])
        pltpu.async_copy(vx_hbm.at[dstv[b]], vbufs[b], gsems[b])

    def gathers_wait(b):
        pltpu.make_async_copy(u_hbm.at[srcv[b]], ubufs[b], gsems[b]).wait()
        pltpu.make_async_copy(vx_hbm.at[dstv[b]], vbufs[b], gsems[b]).wait()

    pltpu.sync_copy(src_hbm.at[pl.ds(base0, _CH)], s0_src)
    pltpu.sync_copy(dst_hbm.at[pl.ds(base0, _CH)], s0_dst)
    pltpu.sync_copy(src_hbm.at[pl.ds(base0 + _CH, _CH)], s1_src)
    pltpu.sync_copy(dst_hbm.at[pl.ds(base0 + _CH, _CH)], s1_dst)
    gathers_issue(0)
    gathers_issue(1)
    idx_issue(2, 2)

    def outer(it, carry):
        for b in range(_NBUF):
            i = it * _NBUF + b
            nb2 = (b + 2) % _NBUF

            @pl.when(i >= _NBUF)
            def _():
                pltpu.make_async_copy(mbufs[b], aggs.at[dws[b]],
                                      ssems[b]).wait()

            @pl.when(i < _NCHUNK)
            def _():
                gathers_wait(b)
                for off in (0, 16, _CH - 16):
                    dws[b][pl.ds(off, 16)] = dstv[b][pl.ds(off, 16)]

            @pl.when(i + _NBUF < _NCHUNK)
            def _():
                idx_issue(i + _NBUF, b)

            @pl.when(i + 2 < _NCHUNK)
            def _():
                idx_wait(nb2)
                gathers_issue(nb2)

            @pl.when(i < _NCHUNK)
            def _():
                def row(r, rc):
                    for j in range(_D // 16):
                        a = ubufs[b][r, pl.ds(j * 16, 16)]
                        v = vbufs[b][r, pl.ds(j * 16, 16)]
                        mbufs[b][r, pl.ds(j * 16, 16)] = jnp.maximum(
                            a + v, 0.0)
                    return rc
                lax.fori_loop(0, _CH, row, 0)
                pltpu.async_copy(mbufs[b], aggs.at[dws[b]], ssems[b],
                                 add=True)
        return carry
    lax.fori_loop(0, _NOUTER, outer, 0)

    for b in range(_NCHUNK - (_NOUTER - 1) * _NBUF):
        pltpu.make_async_copy(mbufs[b], aggs.at[dws[b]], ssems[b]).wait()

    plsc.subcore_barrier()

    def wcopy(k, carry):
        blk = s + _NS * k

        @pl.when(blk < _NRB)
        def _():
            rows = pl.ds(blk * _RB, _RB)
            pltpu.sync_copy(aggs.at[rows], agg_hbm.at[c, rows])
        return carry
    lax.fori_loop(0, (_NRB + _NS - 1) // _NS, wcopy, 0)


def _make_edge_kernel():
    mesh = plsc.VectorSubcoreMesh(core_axis_name="c", subcore_axis_name="s")
    slot = [
        pltpu.VMEM((_CH,), jnp.int32),
        pltpu.VMEM((_CH,), jnp.int32),
        pltpu.VMEM((_CH,), jnp.int32),
        pltpu.VMEM((_CH, _D), jnp.float32),
        pltpu.VMEM((_CH, _D), jnp.float32),
        pltpu.VMEM((_CH, _D), jnp.float32),
    ]
    return pl.kernel(
        _edge_body,
        out_type=jax.ShapeDtypeStruct((_NC, _N, _D), jnp.float32),
        mesh=mesh,
        scratch_types=[
            *slot, *slot, *slot,
            pltpu.VMEM_SHARED((_N, _D), jnp.float32),
            pltpu.SemaphoreType.DMA,
            pltpu.SemaphoreType.DMA,
            pltpu.SemaphoreType.DMA,
            pltpu.SemaphoreType.DMA,
            pltpu.SemaphoreType.DMA,
            pltpu.SemaphoreType.DMA,
            pltpu.SemaphoreType.DMA,
            pltpu.SemaphoreType.DMA,
            pltpu.SemaphoreType.DMA,
        ],
    )


def _embed_body(xidx_ref, emb_ref, vx_ref, u_ref):
    ids = xidx_ref[...]
    io = lax.broadcasted_iota(jnp.int32, (_N, _A), 1)
    oh = jnp.where(io == ids, 1.0, 0.0).astype(jnp.float32)
    vx = jnp.dot(oh, emb_ref[...], preferred_element_type=jnp.float32)
    vx_ref[...] = vx
    u_ref[...] = vx * 2.0


def _bn_block(y, g, b):
    m = jnp.mean(y, axis=0, keepdims=True)
    yc = y - m
    v = jnp.mean(yc * yc, axis=0, keepdims=True)
    return yc * lax.rsqrt(v + 1e-5) * g + b


def _mlp_body(x_ref, agg_ref, vx_ref, w1_ref, b1_ref, g1_ref, t1_ref,
              w2_ref, b2_ref, g2_ref, t2_ref, xo_ref, uo_ref):
    h = x_ref[...] + agg_ref[0] + agg_ref[1]
    y = jnp.dot(h, w1_ref[...], preferred_element_type=jnp.float32)
    y = jnp.maximum(_bn_block(y + b1_ref[...], g1_ref[...], t1_ref[...]), 0.0)
    z = jnp.dot(y, w2_ref[...], preferred_element_type=jnp.float32)
    z = jnp.maximum(_bn_block(z + b2_ref[...], g2_ref[...], t2_ref[...]), 0.0)
    xo_ref[...] = z
    uo_ref[...] = z + vx_ref[...]


def _pool_body(x_ref, batch_ref, w1_ref, b1_ref, w2_ref, b2_ref, out_ref):
    bt = batch_ref[...]
    io = lax.broadcasted_iota(jnp.int32, (_B, _N), 0)
    oh = jnp.where(io == bt, 1.0, 0.0).astype(jnp.float32)
    pooled = jnp.dot(oh, x_ref[...], preferred_element_type=jnp.float32)
    hh = jnp.maximum(
        jnp.dot(pooled, w1_ref[...], preferred_element_type=jnp.float32)
        + b1_ref[...], 0.0)
    out_ref[...] = (
        jnp.dot(hh, w2_ref[...], preferred_element_type=jnp.float32)
        + b2_ref[...])


@jax.jit
def kernel(x_idx, edge_index, batch, emb, convs, lin1_W, lin1_b, lin2_W,
           lin2_b):
    f32 = jnp.float32
    embed = pl.pallas_call(
        _embed_body,
        out_shape=[jax.ShapeDtypeStruct((_N, _D), f32),
                   jax.ShapeDtypeStruct((_N, _D), f32)],
    )
    vx, u = embed(x_idx, emb)

    edge_fn = _make_edge_kernel()
    mlp = pl.pallas_call(
        _mlp_body,
        out_shape=[jax.ShapeDtypeStruct((_N, _D), f32),
                   jax.ShapeDtypeStruct((_N, _D), f32)],
    )

    src = edge_index[0]
    dst = edge_index[1]
    x = vx
    for p in convs:
        aggs = edge_fn(src, dst, u, vx)
        x, u = mlp(x, aggs, vx,
                   p['W1'], p['b1'].reshape(1, -1), p['g1'].reshape(1, -1),
                   p['bt1'].reshape(1, -1),
                   p['W2'], p['b2'].reshape(1, -1), p['g2'].reshape(1, -1),
                   p['bt2'].reshape(1, -1))

    pool = pl.pallas_call(
        _pool_body,
        out_shape=jax.ShapeDtypeStruct((_B, _OUT), f32),
    )
    return pool(x, batch.reshape(1, -1), lin1_W, lin1_b.reshape(1, -1),
                lin2_W, lin2_b.reshape(1, -1))
